# baseline (speedup 1.0000x reference)
import functools

import jax
import jax.numpy as jnp
from jax import lax
from jax.experimental import pallas as pl
from jax.experimental.pallas import tpu as pltpu
from jax.experimental.pallas import tpu_sc as plsc

PH = 7
PW = 32
HSS = 0.0625
WSS = 0.25
B, C, H, W = 4, 128, 48, 160
NROIS = 1000
NCELL = NROIS * PH * PW
NSTRIPE = NROIS * PH
NSLABS = 16
KH_BASE = (0, 1, 3, 7)
ZSTRIPE = 15 * H
NW = 32
SPW = 219
NSTRIPE_PAD = NW * SPW
NB = 3
NGROUPS = SPW // NB
PROW = 80
SD = W * C
WSPLIT = 4
W4 = W // WSPLIT


def _round(x):
    return jnp.floor(x + 0.5)


def _transpose_body(feats_ref, out_ref):
    x = feats_ref[0].reshape(C, H * W)
    out_ref[...] = x.T


def _pairs_body(in_ref, out_ref):
    t00 = in_ref[0]

    def roll(a, d):
        return jnp.concatenate([a[d:], a[:d]], axis=0)

    t10 = jnp.maximum(t00, roll(t00, 1))
    t20 = jnp.maximum(t10, roll(t10, 2))
    t30 = jnp.maximum(t20, roll(t20, 4))
    chain = (t00, t10, t20, t30)
    for kh in range(4):
        tb = chain[kh]
        out_ref[KH_BASE[kh]] = tb
        for d in range(1, 1 << kh):
            out_ref[KH_BASE[kh] + d] = jnp.maximum(tb, roll(tb, d))
    out_ref[15] = jnp.zeros((H, W4, C), jnp.float32)


def _build_table(feats):
    t2d = pl.pallas_call(
        _transpose_body,
        grid=(B,),
        in_specs=[pl.BlockSpec((1, C, H, W), lambda b: (b, 0, 0, 0))],
        out_specs=pl.BlockSpec((H * W, C), lambda b: (b * NSLABS, 0)),
        out_shape=jax.ShapeDtypeStruct((B * NSLABS * H * W, C), jnp.float32),
    )(feats)
    t4d = t2d.reshape(B * NSLABS, H, W, C)
    out = pl.pallas_call(
        _pairs_body,
        grid=(B, WSPLIT),
        in_specs=[pl.BlockSpec((1, H, W4, C), lambda b, w: (b * NSLABS, 0, w, 0))],
        out_specs=pl.BlockSpec((NSLABS, H, W4, C), lambda b, w: (b, 0, w, 0)),
        out_shape=jax.ShapeDtypeStruct(t4d.shape, t4d.dtype),
        input_output_aliases={0: 0},
    )(t4d)
    return out.reshape(B * NSLABS * H, SD)


def _stripe_params(rois):
    bind = rois[:, 0].astype(jnp.int32)
    rsw = _round(rois[:, 1] * WSS).astype(jnp.int32)
    rsh = _round(rois[:, 2] * HSS).astype(jnp.int32)
    rew = _round(rois[:, 3] * WSS).astype(jnp.int32)
    reh = _round(rois[:, 4] * HSS).astype(jnp.int32)
    roi_w = jnp.maximum(rew - rsw + 1, 1)
    roi_h = jnp.maximum(reh - rsh + 1, 1)
    rois_pw = jnp.ceil((PH * roi_w).astype(jnp.float32)
                       / roi_h.astype(jnp.float32)).astype(jnp.int32)
    rois_pw = jnp.maximum(rois_pw, 1)
    bin_h = roi_h.astype(jnp.float32) / float(PH)
    bin_w = roi_w.astype(jnp.float32) / rois_pw.astype(jnp.float32)
    ph = jnp.arange(PH, dtype=jnp.float32)
    pw = jnp.arange(PW, dtype=jnp.float32)
    hstart = jnp.clip(jnp.floor(ph[None, :] * bin_h[:, None]).astype(jnp.int32)
                      + rsh[:, None], 0, H)
    hend = jnp.clip(jnp.ceil((ph[None, :] + 1.0) * bin_h[:, None]).astype(jnp.int32)
                    + rsh[:, None], 0, H)
    wstart = jnp.clip(jnp.floor(pw[None, :] * bin_w[:, None]).astype(jnp.int32)
                      + rsw[:, None], 0, W)
    wend = jnp.clip(jnp.ceil((pw[None, :] + 1.0) * bin_w[:, None]).astype(jnp.int32)
                    + rsw[:, None], 0, W)
    skip = wstart >= rew[:, None]
    hlen = hend - hstart
    wlen = wend - wstart
    live_ph = hlen > 0
    kh = ((hlen >= 2).astype(jnp.int32) + (hlen >= 4).astype(jnp.int32)
          + (hlen >= 8).astype(jnp.int32))
    dh = jnp.clip(hend - (1 << kh) - hstart, 0, 7)
    kh_base = jnp.array(KH_BASE, jnp.int32)[kh]
    slab = kh_base + dh
    srow = (bind[:, None] * NSLABS + slab) * H + hstart
    srow = jnp.where(live_ph, srow, ZSTRIPE)
    cell_live = (live_ph[:, :, None] & (wlen[:, None, :] > 0)
                 & (~skip[:, None, :]))
    wlen_eff = jnp.where(cell_live, wlen[:, None, :], 0)
    ws_b = jnp.broadcast_to(wstart[:, None, :], (NROIS, PH, PW))
    npad = NSTRIPE_PAD - NSTRIPE
    col0 = jnp.concatenate([srow.reshape(-1),
                            jnp.full((npad,), ZSTRIPE, jnp.int32)])
    ws_p = jnp.concatenate([ws_b.reshape(NSTRIPE, PW),
                            jnp.zeros((npad, PW), jnp.int32)])
    wl_p = jnp.concatenate([wlen_eff.reshape(NSTRIPE, PW),
                            jnp.zeros((npad, PW), jnp.int32)])
    params = jnp.concatenate(
        [col0[:, None], jnp.zeros((NSTRIPE_PAD, 15), jnp.int32), ws_p, wl_p],
        axis=1)
    return params.reshape(-1)


NEG = -1e37


def _sc_body(table_hbm, params_hbm, out_hbm, *scr):
    pv = scr[0]
    sbuf = scr[1:1 + NB]
    obuf = scr[1 + NB:1 + 2 * NB]
    gsem = scr[1 + 2 * NB:1 + 3 * NB]
    osem = scr[1 + 3 * NB:1 + 4 * NB]
    nc = 2
    wid = lax.axis_index("s") * nc + lax.axis_index("c")
    pltpu.sync_copy(params_hbm.at[pl.ds(wid * SPW * PROW, SPW * PROW)], pv)

    def fire(t, s):
        pltpu.async_copy(table_hbm.at[pv.at[pl.ds(t * PROW, 1)]],
                         sbuf[s], gsem[s])

    for s in range(NB):
        fire(s, s)

    def group_body(g, carry):
        for s in range(NB):
            t = g * NB + s
            stripe = wid * SPW + t
            pltpu.make_async_copy(table_hbm.at[pv.at[pl.ds(t * PROW, 1)]],
                                  sbuf[s], gsem[s]).wait()

            @pl.when(t >= NB)
            def _():
                pltpu.make_async_copy(
                    obuf[s], out_hbm.at[pl.ds(0, PW), :], osem[s]).wait()

            ws_g = [pv[pl.ds(t * PROW + 16, 16)], pv[pl.ds(t * PROW + 32, 16)]]
            wl_g = [pv[pl.ds(t * PROW + 48, 16)], pv[pl.ds(t * PROW + 64, 16)]]
            for cell in range(PW):
                grp, ln = divmod(cell, 16)
                ws_c = ws_g[grp][ln]
                wl_c = wl_g[grp][ln]

                def rbody(r, acc):
                    off = (ws_c + r) * C
                    return tuple(
                        jnp.maximum(acc[j], sbuf[s][0, pl.ds(off + j * 16, 16)])
                        for j in range(C // 16))

                acc0 = tuple(jnp.full((16,), NEG, jnp.float32)
                             for _ in range(C // 16))
                acc = lax.fori_loop(0, wl_c, rbody, acc0)

                @pl.when(wl_c > 0)
                def _():
                    for j in range(C // 16):
                        obuf[s][cell, pl.ds(j * 16, 16)] = acc[j]

                @pl.when(wl_c == 0)
                def _():
                    for j in range(C // 16):
                        obuf[s][cell, pl.ds(j * 16, 16)] = jnp.zeros(
                            (16,), jnp.float32)

            pltpu.async_copy(obuf[s], out_hbm.at[pl.ds(stripe * PW, PW), :],
                             osem[s])

            @pl.when(t + NB < SPW)
            def _():
                fire(t + NB, s)
        return carry

    lax.fori_loop(0, NGROUPS, group_body, 0)
    for s in range(NB):
        pltpu.make_async_copy(obuf[s], out_hbm.at[pl.ds(0, PW), :],
                              osem[s]).wait()


def _sc_stripe_pool(table_rows, params):
    mesh = plsc.VectorSubcoreMesh(core_axis_name="c", subcore_axis_name="s")
    scratch = ([pltpu.VMEM((SPW * PROW,), jnp.int32)]
               + [pltpu.VMEM((1, SD), jnp.float32) for _ in range(NB)]
               + [pltpu.VMEM((PW, C), jnp.float32) for _ in range(NB)]
               + [pltpu.SemaphoreType.DMA for _ in range(2 * NB)])
    fn = pl.kernel(
        _sc_body,
        mesh=mesh,
        out_type=jax.ShapeDtypeStruct((NSTRIPE_PAD * PW, C), jnp.float32),
        scratch_types=scratch,
    )
    return fn(table_rows, params)


def _out_transpose_body(in_ref, out_ref):
    rb = out_ref.shape[0]
    x = in_ref[...].reshape(rb, PH * PW, C)
    out_ref[...] = jnp.transpose(x, (0, 2, 1))


def _out_transpose(cells):
    rb = 4
    y = pl.pallas_call(
        _out_transpose_body,
        grid=(NROIS // rb,),
        in_specs=[pl.BlockSpec((rb * PH * PW, C), lambda r: (r, 0))],
        out_specs=pl.BlockSpec((rb, C, PH * PW), lambda r: (r, 0, 0)),
        out_shape=jax.ShapeDtypeStruct((NROIS, C, PH * PW), jnp.float32),
    )(cells)
    return y.reshape(NROIS, C, PH, PW)


@jax.jit
def kernel(np_features, np_rois):
    table_rows = _build_table(np_features)
    params = _stripe_params(np_rois)
    cells = _sc_stripe_pool(table_rows, params)
    return _out_transpose(cells)


# speedup vs baseline: 1.1153x; 1.1153x over previous
import functools

import jax
import jax.numpy as jnp
from jax import lax
from jax.experimental import pallas as pl
from jax.experimental.pallas import tpu as pltpu
from jax.experimental.pallas import tpu_sc as plsc

PH = 7
PW = 32
HSS = 0.0625
WSS = 0.25
B, C, H, W = 4, 128, 48, 160
NROIS = 1000
NCELL = NROIS * PH * PW
NSTRIPE = NROIS * PH
NSLABS = 16
KH_BASE = (0, 1, 3, 7)
ZSTRIPE = 15 * H
NW = 32
SPW = 219
NSTRIPE_PAD = NW * SPW
NB = 3
NGROUPS = SPW // NB
PROW = 80
SD = W * C
WSPLIT = 4
W4 = W // WSPLIT


def _round(x):
    return jnp.floor(x + 0.5)


def _transpose_body(feats_ref, out_ref):
    x = feats_ref[0].reshape(C, H * W)
    out_ref[...] = x.T


def _pairs_body(in_ref, out_ref):
    t00 = in_ref[0]

    def roll(a, d):
        return jnp.concatenate([a[d:], a[:d]], axis=0)

    t10 = jnp.maximum(t00, roll(t00, 1))
    t20 = jnp.maximum(t10, roll(t10, 2))
    t30 = jnp.maximum(t20, roll(t20, 4))
    chain = (t00, t10, t20, t30)
    for kh in range(4):
        tb = chain[kh]
        out_ref[KH_BASE[kh]] = tb
        for d in range(1, 1 << kh):
            out_ref[KH_BASE[kh] + d] = jnp.maximum(tb, roll(tb, d))
    out_ref[15] = jnp.zeros((H, W4, C), jnp.float32)


def _build_table(feats):
    t2d = pl.pallas_call(
        _transpose_body,
        grid=(B,),
        in_specs=[pl.BlockSpec((1, C, H, W), lambda b: (b, 0, 0, 0))],
        out_specs=pl.BlockSpec((H * W, C), lambda b: (b * NSLABS, 0)),
        out_shape=jax.ShapeDtypeStruct((B * NSLABS * H * W, C), jnp.float32),
    )(feats)
    t4d = t2d.reshape(B * NSLABS, H, W, C)
    out = pl.pallas_call(
        _pairs_body,
        grid=(B, WSPLIT),
        in_specs=[pl.BlockSpec((1, H, W4, C), lambda b, w: (b * NSLABS, 0, w, 0))],
        out_specs=pl.BlockSpec((NSLABS, H, W4, C), lambda b, w: (b, 0, w, 0)),
        out_shape=jax.ShapeDtypeStruct(t4d.shape, t4d.dtype),
        input_output_aliases={0: 0},
    )(t4d)
    return out.reshape(B * NSLABS * H, SD)


def _stripe_params(rois):
    bind = rois[:, 0].astype(jnp.int32)
    rsw = _round(rois[:, 1] * WSS).astype(jnp.int32)
    rsh = _round(rois[:, 2] * HSS).astype(jnp.int32)
    rew = _round(rois[:, 3] * WSS).astype(jnp.int32)
    reh = _round(rois[:, 4] * HSS).astype(jnp.int32)
    roi_w = jnp.maximum(rew - rsw + 1, 1)
    roi_h = jnp.maximum(reh - rsh + 1, 1)
    rois_pw = jnp.ceil((PH * roi_w).astype(jnp.float32)
                       / roi_h.astype(jnp.float32)).astype(jnp.int32)
    rois_pw = jnp.maximum(rois_pw, 1)
    bin_h = roi_h.astype(jnp.float32) / float(PH)
    bin_w = roi_w.astype(jnp.float32) / rois_pw.astype(jnp.float32)
    ph = jnp.arange(PH, dtype=jnp.float32)
    pw = jnp.arange(PW, dtype=jnp.float32)
    hstart = jnp.clip(jnp.floor(ph[None, :] * bin_h[:, None]).astype(jnp.int32)
                      + rsh[:, None], 0, H)
    hend = jnp.clip(jnp.ceil((ph[None, :] + 1.0) * bin_h[:, None]).astype(jnp.int32)
                    + rsh[:, None], 0, H)
    wstart = jnp.clip(jnp.floor(pw[None, :] * bin_w[:, None]).astype(jnp.int32)
                      + rsw[:, None], 0, W)
    wend = jnp.clip(jnp.ceil((pw[None, :] + 1.0) * bin_w[:, None]).astype(jnp.int32)
                    + rsw[:, None], 0, W)
    skip = wstart >= rew[:, None]
    hlen = hend - hstart
    wlen = wend - wstart
    live_ph = hlen > 0
    kh = ((hlen >= 2).astype(jnp.int32) + (hlen >= 4).astype(jnp.int32)
          + (hlen >= 8).astype(jnp.int32))
    dh = jnp.clip(hend - (1 << kh) - hstart, 0, 7)
    kh_base = jnp.array(KH_BASE, jnp.int32)[kh]
    slab = kh_base + dh
    srow = (bind[:, None] * NSLABS + slab) * H + hstart
    srow = jnp.where(live_ph, srow, ZSTRIPE)
    cell_live = (live_ph[:, :, None] & (wlen[:, None, :] > 0)
                 & (~skip[:, None, :]))
    wlen_eff = jnp.where(cell_live, wlen[:, None, :], 0)
    ws_b = jnp.broadcast_to(wstart[:, None, :], (NROIS, PH, PW))
    npad = NSTRIPE_PAD - NSTRIPE
    col0 = jnp.concatenate([srow.reshape(-1),
                            jnp.full((npad,), ZSTRIPE, jnp.int32)])
    ws_p = jnp.concatenate([ws_b.reshape(NSTRIPE, PW),
                            jnp.zeros((npad, PW), jnp.int32)])
    wl_p = jnp.concatenate([wlen_eff.reshape(NSTRIPE, PW),
                            jnp.zeros((npad, PW), jnp.int32)])
    params = jnp.concatenate(
        [col0[:, None], jnp.zeros((NSTRIPE_PAD, 15), jnp.int32), ws_p, wl_p],
        axis=1)
    return params.reshape(-1)


NEG = -1e37


def _sc_body(table_hbm, params_hbm, out_hbm, *scr):
    pv = scr[0]
    sbuf = scr[1:1 + NB]
    obuf = scr[1 + NB:1 + 2 * NB]
    gsem = scr[1 + 2 * NB:1 + 3 * NB]
    osem = scr[1 + 3 * NB:1 + 4 * NB]
    nc = 2
    wid = lax.axis_index("s") * nc + lax.axis_index("c")
    pltpu.sync_copy(params_hbm.at[pl.ds(wid * SPW * PROW, SPW * PROW)], pv)

    def fire(t, s):
        pltpu.async_copy(table_hbm.at[pv.at[pl.ds(t * PROW, 1)]],
                         sbuf[s], gsem[s])

    for s in range(NB):
        fire(s, s)

    def group_body(g, carry):
        for s in range(NB):
            t = g * NB + s
            stripe = wid * SPW + t
            pltpu.make_async_copy(table_hbm.at[pv.at[pl.ds(t * PROW, 1)]],
                                  sbuf[s], gsem[s]).wait()

            @pl.when(t >= NB)
            def _():
                pltpu.make_async_copy(
                    obuf[s], out_hbm.at[pl.ds(0, PW), :], osem[s]).wait()

            ws_g = [pv[pl.ds(t * PROW + 16, 16)], pv[pl.ds(t * PROW + 32, 16)]]
            wl_g = [pv[pl.ds(t * PROW + 48, 16)], pv[pl.ds(t * PROW + 64, 16)]]
            for cell in range(PW):
                grp, ln = divmod(cell, 16)
                ws_c = ws_g[grp][ln]
                wl_c = wl_g[grp][ln]

                def rbody(r, acc):
                    off = (ws_c + r) * C
                    return tuple(
                        jnp.maximum(acc[j], sbuf[s][0, pl.ds(off + j * 16, 16)])
                        for j in range(C // 16))

                acc0 = tuple(jnp.full((16,), NEG, jnp.float32)
                             for _ in range(C // 16))
                acc = lax.fori_loop(0, wl_c, rbody, acc0)
                gate = jnp.broadcast_to(
                    jnp.minimum(wl_c, 1).astype(jnp.float32), (16,))
                for j in range(C // 16):
                    obuf[s][cell, pl.ds(j * 16, 16)] = acc[j] * gate

            pltpu.async_copy(obuf[s], out_hbm.at[pl.ds(stripe * PW, PW), :],
                             osem[s])

            @pl.when(t + NB < SPW)
            def _():
                fire(t + NB, s)
        return carry

    lax.fori_loop(0, NGROUPS, group_body, 0)
    for s in range(NB):
        pltpu.make_async_copy(obuf[s], out_hbm.at[pl.ds(0, PW), :],
                              osem[s]).wait()


def _sc_stripe_pool(table_rows, params):
    mesh = plsc.VectorSubcoreMesh(core_axis_name="c", subcore_axis_name="s")
    scratch = ([pltpu.VMEM((SPW * PROW,), jnp.int32)]
               + [pltpu.VMEM((1, SD), jnp.float32) for _ in range(NB)]
               + [pltpu.VMEM((PW, C), jnp.float32) for _ in range(NB)]
               + [pltpu.SemaphoreType.DMA for _ in range(2 * NB)])
    fn = pl.kernel(
        _sc_body,
        mesh=mesh,
        out_type=jax.ShapeDtypeStruct((NSTRIPE_PAD * PW, C), jnp.float32),
        scratch_types=scratch,
    )
    return fn(table_rows, params)


def _out_transpose_body(in_ref, out_ref):
    rb = out_ref.shape[0]
    x = in_ref[...].reshape(rb, PH * PW, C)
    out_ref[...] = jnp.transpose(x, (0, 2, 1))


def _out_transpose(cells):
    rb = 4
    y = pl.pallas_call(
        _out_transpose_body,
        grid=(NROIS // rb,),
        in_specs=[pl.BlockSpec((rb * PH * PW, C), lambda r: (r, 0))],
        out_specs=pl.BlockSpec((rb, C, PH * PW), lambda r: (r, 0, 0)),
        out_shape=jax.ShapeDtypeStruct((NROIS, C, PH * PW), jnp.float32),
    )(cells)
    return y.reshape(NROIS, C, PH, PW)


@jax.jit
def kernel(np_features, np_rois):
    table_rows = _build_table(np_features)
    params = _stripe_params(np_rois)
    cells = _sc_stripe_pool(table_rows, params)
    return _out_transpose(cells)
